# Initial kernel scaffold; baseline (speedup 1.0000x reference)
import jax
import jax.numpy as jnp
from jax.experimental import pallas as pl


def kernel(points, qff_vector, qff_plane, freqs):
    raise NotImplementedError("write your pallas kernel here")


# speedup vs baseline: 4.6860x; 4.6860x over previous
import functools

import jax
import jax.numpy as jnp
from jax import lax
from jax.experimental import pallas as pl
from jax.experimental.pallas import tpu as pltpu
from jax.experimental.pallas import tpu_sc as plsc

_NF = 6
_Q = 128
_NB = _NF * 6
_PLANE_ROWS = _NB * _Q * _Q
_G = 40
_GP = 40
_VW = 48
_W = 240
_VTW = 73856
_P = 4
_NWORK = 32


def _sc_gather_kernel(N):
    ppw = N // _NWORK
    nchunks = ppw // _P
    mesh = plsc.VectorSubcoreMesh(core_axis_name="c", subcore_axis_name="s")

    @functools.partial(
        pl.kernel,
        mesh=mesh,
        compiler_params=pltpu.CompilerParams(needs_layout_passes=False),
        out_type=jax.ShapeDtypeStruct((N * 48,), jnp.float32),
        scratch_types=[
            pltpu.VMEM((_P, _GP), jnp.int32),
            pltpu.VMEM((_P, _VW), jnp.int32),
            pltpu.VMEM((_P, _W), jnp.float32),
            pltpu.VMEM((_VTW,), jnp.int32),
            pltpu.VMEM((_P, _G, 128), jnp.float32),
            pltpu.VMEM((_P * 48,), jnp.float32),
            pltpu.VMEM((32,), jnp.float32),
            pltpu.VMEM((32,), jnp.float32),
            pltpu.SemaphoreType.DMA,
        ],
    )
    def k(idx_hbm, vr_hbm, wt_hbm, vtab_hbm, tab_hbm, out_hbm,
          idx_v, vr_v, wt_v, vtab_v, rows_v, out_v, fold_v, pack_v, sem):
        wid = lax.axis_index("s") * 2 + lax.axis_index("c")
        base = wid * ppw
        pltpu.sync_copy(vtab_hbm, vtab_v)

        def chunk_body(kk, carry):
            cb = base + kk * _P
            pltpu.sync_copy(idx_hbm.at[pl.ds(cb, _P)], idx_v)
            pltpu.sync_copy(vr_hbm.at[pl.ds(cb, _P)], vr_v)
            pltpu.sync_copy(wt_hbm.at[pl.ds(cb, _P)], wt_v)
            handles = [
                pltpu.async_copy(
                    tab_hbm.at[idx_v.at[p]], rows_v.at[p], sem)
                for p in range(_P)
            ]
            for h in handles:
                h.wait()
            for p in range(_P):
                vrs = [vr_v[p, pl.ds(g * 16, 16)] for g in range(3)]
                for fs in range(12):
                    acc_lo = jnp.zeros((16,), jnp.float32)
                    acc_hi = jnp.zeros((16,), jnp.float32)
                    for a in range(3):
                        b = fs * 3 + a
                        wvec = wt_v[p, pl.ds(b * 6, 16)]
                        w00 = wvec[0]
                        w01 = wvec[1]
                        w10 = wvec[2]
                        w11 = wvec[3]
                        wv0 = wvec[4]
                        wv1 = wvec[5]
                        m_lo = (w00 * rows_v[p, b, pl.ds(0, 16)]
                                + w01 * rows_v[p, b, pl.ds(32, 16)]
                                + w10 * rows_v[p, b, pl.ds(64, 16)]
                                + w11 * rows_v[p, b, pl.ds(96, 16)])
                        m_hi = (w00 * rows_v[p, b, pl.ds(16, 16)]
                                + w01 * rows_v[p, b, pl.ds(48, 16)]
                                + w10 * rows_v[p, b, pl.ds(80, 16)]
                                + w11 * rows_v[p, b, pl.ds(112, 16)])
                        vn = vrs[b // 16][b % 16]
                        pk0 = plsc.bitcast(
                            vtab_v[pl.ds(vn, 16)], jnp.bfloat16)
                        pk1 = plsc.bitcast(
                            vtab_v[pl.ds(vn + 16, 16)], jnp.bfloat16)
                        lo0, hi0 = plsc.unpack(
                            pk0, format=plsc.PackFormat.INTERLEAVED)
                        lo1, hi1 = plsc.unpack(
                            pk1, format=plsc.PackFormat.INTERLEAVED)
                        v_lo = wv0 * lo0 + wv1 * lo1
                        v_hi = wv0 * hi0 + wv1 * hi1
                        acc_lo = acc_lo + m_lo * v_lo
                        acc_hi = acc_hi + m_hi * v_hi
                    s = acc_lo + acc_hi
                    fold_v[pl.ds(0, 16)] = s
                    fold_v[pl.ds(16, 16)] = s
                    t = s + fold_v[pl.ds(4, 16)]
                    fold_v[pl.ds(0, 16)] = t
                    fold_v[pl.ds(16, 16)] = t
                    u = t + fold_v[pl.ds(8, 16)]
                    pack_v[pl.ds((fs % 4) * 4, 16)] = u
                    if fs % 4 == 3:
                        out_v[pl.ds(p * 48 + (fs // 4) * 16, 16)] = (
                            pack_v[pl.ds(0, 16)])
            pltpu.sync_copy(out_v, out_hbm.at[pl.ds(cb * 48, _P * 48)])
            return carry

        lax.fori_loop(0, nchunks, chunk_body, 0)

    return k


def kernel(points, qff_vector, qff_plane, freqs):
    N = points.shape[0]
    f32 = jnp.float32

    fp = points[:, None, :] * freqs[None, :, None]
    enc = jnp.stack([jnp.sin(fp), jnp.cos(fp)], axis=2)

    xe = enc[..., 0]
    ye = enc[..., 1]
    ze = enc[..., 2]
    gx = jnp.stack([ye, xe, xe], axis=-1).reshape(N, _NB)
    gy = jnp.stack([ze, ze, ye], axis=-1).reshape(N, _NB)
    ev = enc.reshape(N, _NB)

    half = (_Q - 1) * 0.5
    ix = (gx + 1.0) * half
    iy = (gy + 1.0) * half
    iv = (ev + 1.0) * half
    x0f = jnp.floor(ix)
    y0f = jnp.floor(iy)
    v0f = jnp.floor(iv)
    wx1 = ix - x0f
    wy1 = iy - y0f
    wv1 = iv - v0f
    wx0 = 1.0 - wx1
    wy0 = 1.0 - wy1
    wv0 = 1.0 - wv1
    x0 = x0f.astype(jnp.int32)
    y0 = y0f.astype(jnp.int32)
    v0 = v0f.astype(jnp.int32)

    bix = jnp.arange(_NB, dtype=jnp.int32)[None, :]
    pidx = (bix * _Q + y0) * _Q + x0
    idx = jnp.concatenate(
        [pidx, jnp.zeros((N, _GP - _NB), jnp.int32)], axis=1)
    vrow = (bix * _Q + v0) * 16
    vr = jnp.concatenate(
        [vrow, jnp.zeros((N, _VW - _NB), jnp.int32)], axis=1)

    wts = jnp.stack(
        [wy0 * wx0, wy0 * wx1, wy1 * wx0, wy1 * wx1, wv0, wv1],
        axis=-1).reshape(N, 6 * _NB).astype(f32)
    wts = jnp.concatenate(
        [wts, jnp.zeros((N, _W - 6 * _NB), f32)], axis=1)

    j = jnp.arange(32)
    perm = (j % 4) * 8 + j // 4
    planeT = qff_plane.transpose(0, 2, 3, 1)[..., perm]
    pp = jnp.pad(planeT, ((0, 0), (0, 1), (0, 1), (0, 0)))
    patch = jnp.concatenate(
        [pp[:, :_Q, :_Q], pp[:, :_Q, 1:], pp[:, 1:, :_Q], pp[:, 1:, 1:]],
        axis=-1).reshape(_PLANE_ROWS, 128)

    vecT = qff_vector[..., 0].transpose(0, 2, 1)[..., perm]
    vflat = jnp.concatenate(
        [vecT.reshape(_NB * _Q, 32), jnp.zeros((1, 32), f32)], axis=0)
    lo_u = lax.bitcast_convert_type(
        vflat[:, :16].astype(jnp.bfloat16), jnp.uint16).astype(jnp.uint32)
    hi_u = lax.bitcast_convert_type(
        vflat[:, 16:].astype(jnp.bfloat16), jnp.uint16).astype(jnp.uint32)
    vtab = lax.bitcast_convert_type(lo_u | (hi_u << 16), jnp.int32)
    vtab = jnp.concatenate(
        [vtab.reshape(-1), jnp.zeros((_VTW - (_NB * _Q + 1) * 16,),
                                     jnp.int32)])

    out = _sc_gather_kernel(N)(idx, vr, wts, vtab, patch)
    return out.reshape(N, 48)
